# baseline (speedup 1.0000x reference)
import dataclasses
import functools

import jax
import jax.numpy as jnp
from jax import lax
from jax.experimental import pallas as pl
from jax.experimental.pallas import tpu as pltpu
from jax.experimental.pallas import tpu_sc as plsc

_N = 10000
_D = 128
_NC = 2
_NS = 16
_NW = _NC * _NS
_L = 16
_HALF = 5120
_ACC_ROWS = 5128
_CHUNK = 64
_EW = 10112
_GQ = _EW // _L
_SK = 160


def _sc_params():
    cp = pltpu.CompilerParams()
    if "needs_layout_passes" in pltpu.CompilerParams.__dataclass_fields__:
        cp = dataclasses.replace(cp, needs_layout_passes=False)
    return cp


def _sc_partition(src_st, dst_st):
    mesh = plsc.VectorSubcoreMesh(core_axis_name="c", subcore_axis_name="s")

    @functools.partial(
        pl.kernel,
        mesh=mesh,
        out_type=(
            jax.ShapeDtypeStruct((_NC, _NC, _NS, 2 * _SK, _CHUNK),
                                 jnp.int32),
            jax.ShapeDtypeStruct((_NC, _NC, _NS, 64), jnp.int32),
        ),
        scratch_types=[
            pltpu.VMEM((_GQ // 8, 128), jnp.int32),
            pltpu.VMEM((_GQ // 8, 128), jnp.int32),
            pltpu.VMEM((2 * _SK, _CHUNK), jnp.int32),
            pltpu.VMEM((2 * _SK, _CHUNK), jnp.int32),
            pltpu.VMEM((2, 64), jnp.int32),
        ],
        compiler_params=_sc_params(),
    )
    def part_kernel(src_hbm, dst_hbm, lists_hbm, cnt_hbm,
                    src_v, dst_v, ol0, ol1, cnt_v):
        cid = lax.axis_index("c")
        sid = lax.axis_index("s")
        pltpu.sync_copy(src_hbm.at[cid, sid], src_v)
        pltpu.sync_copy(dst_hbm.at[cid, sid], dst_v)

        lanes = lax.iota(jnp.int32, _L)
        zeros16 = jnp.zeros((_L,), jnp.int32)
        dump16 = jnp.full((_L,), _HALF, jnp.int32)

        def body(g, carry):
            oa, ob = carry
            r = g >> 3
            q = (g & 7) * _L
            s_ = src_v[r, pl.ds(q, _L)]
            d = dst_v[r, pl.ds(q, _L)]
            ma = d < _HALF
            inca = jnp.cumsum(ma.astype(jnp.int32))
            posa = (oa - 1) + inca
            plsc.store_scatter(ol0, [(posa >> 6) * 2, posa & 63],
                               s_, mask=ma)
            plsc.store_scatter(ol0, [(posa >> 6) * 2 + 1, posa & 63],
                               d, mask=ma)
            mb = d >= _HALF
            incb = jnp.cumsum(mb.astype(jnp.int32))
            posb = (ob - 1) + incb
            plsc.store_scatter(ol1, [(posb >> 6) * 2, posb & 63],
                               s_, mask=mb)
            plsc.store_scatter(ol1, [(posb >> 6) * 2 + 1, posb & 63],
                               d - _HALF, mask=mb)
            return (oa + jnp.max(inca), ob + jnp.max(incb))

        oa, ob = lax.fori_loop(0, _GQ, body,
                               (jnp.int32(0), jnp.int32(0)))

        def fill(o, olr):
            pad = (-o) & (4 * _CHUNK - 1)
            for t in range(4 * _CHUNK // _L):
                base = t * _L + lanes
                m = base < pad
                idx = o + base
                plsc.store_scatter(olr, [(idx >> 6) * 2, idx & 63],
                                   zeros16, mask=m)
                plsc.store_scatter(olr, [(idx >> 6) * 2 + 1, idx & 63],
                                   dump16, mask=m)
            return (o + pad) >> 6

        na = fill(oa, ol0)
        nb = fill(ob, ol1)
        na16 = jnp.full((_L,), 1, jnp.int32) * na
        nb16 = jnp.full((_L,), 1, jnp.int32) * nb
        for t in range(64 // _L):
            cnt_v[0, pl.ds(t * _L, _L)] = na16
            cnt_v[1, pl.ds(t * _L, _L)] = nb16

        pltpu.sync_copy(ol0, lists_hbm.at[0, cid, sid])
        pltpu.sync_copy(ol1, lists_hbm.at[1, cid, sid])
        pltpu.sync_copy(cnt_v.at[0], cnt_hbm.at[0, cid, sid])
        pltpu.sync_copy(cnt_v.at[1], cnt_hbm.at[1, cid, sid])

    return part_kernel(src_st, dst_st)


def _sc_aggregate(X, lists, counts, zeros):
    mesh = plsc.VectorSubcoreMesh(core_axis_name="c", subcore_axis_name="s")

    @functools.partial(
        pl.kernel,
        mesh=mesh,
        out_type=jax.ShapeDtypeStruct((_NC * _HALF, _D), jnp.float32),
        scratch_types=[
            pltpu.VMEM((8, _CHUNK), jnp.int32),
            pltpu.VMEM((_CHUNK, _D), jnp.float32),
            pltpu.VMEM_SHARED((_N, _D), jnp.float32),
            pltpu.VMEM_SHARED((_ACC_ROWS, _D), jnp.float32),
        ],
        compiler_params=_sc_params(),
    )
    def agg_kernel(x_hbm, lists_hbm, cnt_hbm, zeros_hbm, out_hbm,
                   slab_v, rows_v, x_sh, acc_sh):
        cid = lax.axis_index("c")
        sid = lax.axis_index("s")

        pltpu.sync_copy(x_hbm.at[pl.ds(sid * 624, 624)],
                        x_sh.at[pl.ds(sid * 624, 624)])
        pltpu.sync_copy(zeros_hbm.at[pl.ds(sid * 320, 320)],
                        acc_sh.at[pl.ds(sid * 320, 320)])

        @pl.when(sid == 0)
        def _():
            pltpu.sync_copy(x_hbm.at[pl.ds(9984, 16)],
                            x_sh.at[pl.ds(9984, 16)])
            pltpu.sync_copy(zeros_hbm.at[pl.ds(5120, 8)],
                            acc_sh.at[pl.ds(5120, 8)])

        plsc.subcore_barrier()

        for csrc in range(_NC):
            pltpu.sync_copy(cnt_hbm.at[cid, csrc, sid], slab_v.at[0])
            n = jnp.max(slab_v[0, pl.ds(0, _L)])

            @pl.loop(0, n >> 2)
            def _(t):
                pltpu.sync_copy(
                    lists_hbm.at[cid, csrc, sid, pl.ds(t * 8, 8)],
                    slab_v)
                for u in range(4):
                    pltpu.sync_copy(x_sh.at[slab_v.at[2 * u]], rows_v)
                    pltpu.sync_copy(rows_v,
                                    acc_sh.at[slab_v.at[2 * u + 1]],
                                    add=True)

        plsc.subcore_barrier()
        pltpu.sync_copy(
            acc_sh.at[pl.ds(sid * 320, 320)],
            out_hbm.at[pl.ds(cid * _HALF + sid * 320, 320)])

    return agg_kernel(X, lists, counts, zeros)


def _tc_finish(agg, X, W):
    blk = 1000

    def mm_kernel(a_ref, x_ref, w_ref, o_ref):
        o_ref[...] = jnp.dot(a_ref[...] + x_ref[...], w_ref[...],
                             preferred_element_type=jnp.float32)

    row_spec = pl.BlockSpec((blk, _D), lambda i: (i, 0))
    return pl.pallas_call(
        mm_kernel,
        grid=(_N // blk,),
        in_specs=[row_spec, row_spec,
                  pl.BlockSpec((_D, _D), lambda i: (0, 0))],
        out_specs=row_spec,
        out_shape=jax.ShapeDtypeStruct((_N, _D), jnp.float32),
    )(agg, X, W)


def kernel(X, edge_index, W):
    src = edge_index[0]
    dst = edge_index[1]
    E = src.shape[0]

    e_pad = _NW * _EW
    pad = e_pad - E
    src_p = jnp.concatenate([src, jnp.zeros((pad,), jnp.int32)])
    dst_p = jnp.concatenate([dst, jnp.full((pad,), _N, jnp.int32)])
    src_st = src_p.reshape(_NC, _NS, _GQ // 8, 128)
    dst_st = dst_p.reshape(_NC, _NS, _GQ // 8, 128)
    zeros = jnp.zeros((_ACC_ROWS, _D), jnp.float32)

    lists, counts = _sc_partition(src_st, dst_st)
    agg = _sc_aggregate(X, lists, counts, zeros)
    return _tc_finish(agg, X, W)


# speedup vs baseline: 1.0185x; 1.0185x over previous
import dataclasses
import functools

import jax
import jax.numpy as jnp
from jax import lax
from jax.experimental import pallas as pl
from jax.experimental.pallas import tpu as pltpu
from jax.experimental.pallas import tpu_sc as plsc

_N = 10000
_D = 128
_NC = 2
_NS = 16
_NW = _NC * _NS
_L = 16
_HALF = 5120
_ACC_ROWS = 5128
_CHUNK = 64
_EW = 10112
_GQ = _EW // _L
_SK = 160


def _sc_params():
    cp = pltpu.CompilerParams()
    if "needs_layout_passes" in pltpu.CompilerParams.__dataclass_fields__:
        cp = dataclasses.replace(cp, needs_layout_passes=False)
    return cp


def _sc_partition(src_st, dst_st):
    mesh = plsc.VectorSubcoreMesh(core_axis_name="c", subcore_axis_name="s")

    @functools.partial(
        pl.kernel,
        mesh=mesh,
        out_type=(
            jax.ShapeDtypeStruct((_NC, _NC, _NS, 2 * _SK, _CHUNK),
                                 jnp.int32),
            jax.ShapeDtypeStruct((_NC, _NC, _NS, 64), jnp.int32),
        ),
        scratch_types=[
            pltpu.VMEM((_GQ // 8, 128), jnp.int32),
            pltpu.VMEM((_GQ // 8, 128), jnp.int32),
            pltpu.VMEM((2 * _SK, _CHUNK), jnp.int32),
            pltpu.VMEM((2 * _SK, _CHUNK), jnp.int32),
            pltpu.VMEM((2, 64), jnp.int32),
        ],
        compiler_params=_sc_params(),
    )
    def part_kernel(src_hbm, dst_hbm, lists_hbm, cnt_hbm,
                    src_v, dst_v, ol0, ol1, cnt_v):
        cid = lax.axis_index("c")
        sid = lax.axis_index("s")
        pltpu.sync_copy(src_hbm.at[cid, sid], src_v)
        pltpu.sync_copy(dst_hbm.at[cid, sid], dst_v)

        lanes = lax.iota(jnp.int32, _L)
        zeros16 = jnp.zeros((_L,), jnp.int32)
        dump16 = jnp.full((_L,), _HALF, jnp.int32)

        def body(g, carry):
            oa, ob = carry
            r = g >> 3
            q = (g & 7) * _L
            s_ = src_v[r, pl.ds(q, _L)]
            d = dst_v[r, pl.ds(q, _L)]
            ma = d < _HALF
            inca = jnp.cumsum(ma.astype(jnp.int32))
            posa = (oa - 1) + inca
            plsc.store_scatter(ol0, [(posa >> 6) * 2, posa & 63],
                               s_, mask=ma)
            plsc.store_scatter(ol0, [(posa >> 6) * 2 + 1, posa & 63],
                               d, mask=ma)
            mb = d >= _HALF
            incb = jnp.cumsum(mb.astype(jnp.int32))
            posb = (ob - 1) + incb
            plsc.store_scatter(ol1, [(posb >> 6) * 2, posb & 63],
                               s_, mask=mb)
            plsc.store_scatter(ol1, [(posb >> 6) * 2 + 1, posb & 63],
                               d - _HALF, mask=mb)
            return (oa + jnp.max(inca), ob + jnp.max(incb))

        oa, ob = lax.fori_loop(0, _GQ, body,
                               (jnp.int32(0), jnp.int32(0)))

        def fill(o, olr):
            pad = (-o) & (_CHUNK - 1)
            for t in range(_CHUNK // _L):
                base = t * _L + lanes
                m = base < pad
                idx = o + base
                plsc.store_scatter(olr, [(idx >> 6) * 2, idx & 63],
                                   zeros16, mask=m)
                plsc.store_scatter(olr, [(idx >> 6) * 2 + 1, idx & 63],
                                   dump16, mask=m)
            return (o + pad) >> 6

        na = fill(oa, ol0)
        nb = fill(ob, ol1)
        na16 = jnp.full((_L,), 1, jnp.int32) * na
        nb16 = jnp.full((_L,), 1, jnp.int32) * nb
        for t in range(64 // _L):
            cnt_v[0, pl.ds(t * _L, _L)] = na16
            cnt_v[1, pl.ds(t * _L, _L)] = nb16

        pltpu.sync_copy(ol0, lists_hbm.at[0, cid, sid])
        pltpu.sync_copy(ol1, lists_hbm.at[1, cid, sid])
        pltpu.sync_copy(cnt_v.at[0], cnt_hbm.at[0, cid, sid])
        pltpu.sync_copy(cnt_v.at[1], cnt_hbm.at[1, cid, sid])

    return part_kernel(src_st, dst_st)


def _sc_aggregate(X, lists, counts, zeros):
    mesh = plsc.VectorSubcoreMesh(core_axis_name="c", subcore_axis_name="s")

    @functools.partial(
        pl.kernel,
        mesh=mesh,
        out_type=jax.ShapeDtypeStruct((_NC * _HALF, _D), jnp.float32),
        scratch_types=[
            pltpu.VMEM((8, _CHUNK), jnp.int32),
            pltpu.VMEM((_CHUNK, _D), jnp.float32),
            pltpu.VMEM_SHARED((_N, _D), jnp.float32),
            pltpu.VMEM_SHARED((_ACC_ROWS, _D), jnp.float32),
        ],
        compiler_params=_sc_params(),
    )
    def agg_kernel(x_hbm, lists_hbm, cnt_hbm, zeros_hbm, out_hbm,
                   slab_v, rows_v, x_sh, acc_sh):
        cid = lax.axis_index("c")
        sid = lax.axis_index("s")

        pltpu.sync_copy(x_hbm.at[pl.ds(sid * 624, 624)],
                        x_sh.at[pl.ds(sid * 624, 624)])
        pltpu.sync_copy(zeros_hbm.at[pl.ds(sid * 320, 320)],
                        acc_sh.at[pl.ds(sid * 320, 320)])

        @pl.when(sid == 0)
        def _():
            pltpu.sync_copy(x_hbm.at[pl.ds(9984, 16)],
                            x_sh.at[pl.ds(9984, 16)])
            pltpu.sync_copy(zeros_hbm.at[pl.ds(5120, 8)],
                            acc_sh.at[pl.ds(5120, 8)])

        plsc.subcore_barrier()

        for csrc in range(_NC):
            pltpu.sync_copy(cnt_hbm.at[cid, csrc, sid], slab_v.at[0])
            n = jnp.max(slab_v[0, pl.ds(0, _L)])

            @pl.loop(0, n)
            def _(j):
                jm = j & 3

                @pl.when(jm == 0)
                def _():
                    pltpu.sync_copy(
                        lists_hbm.at[cid, csrc, sid,
                                     pl.ds((j >> 2) * 8, 8)],
                        slab_v)

                pltpu.sync_copy(x_sh.at[slab_v.at[2 * jm]], rows_v)
                pltpu.sync_copy(rows_v, acc_sh.at[slab_v.at[2 * jm + 1]],
                                add=True)

        plsc.subcore_barrier()
        pltpu.sync_copy(
            acc_sh.at[pl.ds(sid * 320, 320)],
            out_hbm.at[pl.ds(cid * _HALF + sid * 320, 320)])

    return agg_kernel(X, lists, counts, zeros)


def _tc_finish(agg, X, W):
    blk = 1000

    def mm_kernel(a_ref, x_ref, w_ref, o_ref):
        o_ref[...] = jnp.dot(a_ref[...] + x_ref[...], w_ref[...],
                             preferred_element_type=jnp.float32)

    row_spec = pl.BlockSpec((blk, _D), lambda i: (i, 0))
    return pl.pallas_call(
        mm_kernel,
        grid=(_N // blk,),
        in_specs=[row_spec, row_spec,
                  pl.BlockSpec((_D, _D), lambda i: (0, 0))],
        out_specs=row_spec,
        out_shape=jax.ShapeDtypeStruct((_N, _D), jnp.float32),
    )(agg, X, W)


def kernel(X, edge_index, W):
    src = edge_index[0]
    dst = edge_index[1]
    E = src.shape[0]

    e_pad = _NW * _EW
    pad = e_pad - E
    src_p = jnp.concatenate([src, jnp.zeros((pad,), jnp.int32)])
    dst_p = jnp.concatenate([dst, jnp.full((pad,), _N, jnp.int32)])
    src_st = src_p.reshape(_NC, _NS, _GQ // 8, 128)
    dst_st = dst_p.reshape(_NC, _NS, _GQ // 8, 128)
    zeros = jnp.zeros((_ACC_ROWS, _D), jnp.float32)

    lists, counts = _sc_partition(src_st, dst_st)
    agg = _sc_aggregate(X, lists, counts, zeros)
    return _tc_finish(agg, X, W)
